# baseline (speedup 1.0000x reference)
import functools

import jax
import jax.numpy as jnp
from jax import lax
from jax.experimental import pallas as pl
from jax.experimental.pallas import tpu as pltpu
from jax.experimental.pallas import tpu_sc as plsc

DIM = 64
PAD = 128
LANES = 16
NUM_CORES = 2
NUM_SUBCORES = 16
NUM_WORKERS = NUM_CORES * NUM_SUBCORES
CHUNK = 64


def _sc_gather(total):
    per_w = total // NUM_WORKERS
    n_chunks = per_w // CHUNK
    groups = CHUNK // LANES

    mesh = plsc.VectorSubcoreMesh(
        core_axis_name="c", subcore_axis_name="s",
        num_cores=NUM_CORES, num_subcores=NUM_SUBCORES)

    @functools.partial(
        pl.kernel,
        out_type=jax.ShapeDtypeStruct((total,), jnp.float32),
        mesh=mesh,
        compiler_params=pltpu.CompilerParams(needs_layout_passes=False),
        scratch_types=[
            pltpu.VMEM((n_chunks, CHUNK), jnp.int32),
            pltpu.VMEM((n_chunks, CHUNK), jnp.int32),
            pltpu.VMEM((n_chunks, CHUNK), jnp.int32),
            pltpu.VMEM((n_chunks, CHUNK), jnp.int32),
            pltpu.VMEM((n_chunks, CHUNK), jnp.int32),
            pltpu.VMEM((n_chunks, CHUNK), jnp.int32),
            pltpu.VMEM((CHUNK, PAD), jnp.float32),
            pltpu.VMEM((CHUNK, PAD), jnp.float32),
            pltpu.VMEM((CHUNK, PAD), jnp.float32),
            pltpu.VMEM((CHUNK, PAD), jnp.float32),
            pltpu.VMEM((CHUNK, PAD), jnp.float32),
            pltpu.VMEM((CHUNK, PAD), jnp.float32),
            pltpu.VMEM((per_w,), jnp.float32),
            pltpu.SemaphoreType.DMA,
            pltpu.SemaphoreType.DMA,
        ],
    )
    def k(hrow_hbm, hcol_hbm, rrow_hbm, rcol_hbm, trow_hbm, tcol_hbm,
          ent_hbm, rel_hbm, out_hbm,
          hrow_v, hcol_v, rrow_v, rcol_v, trow_v, tcol_v,
          h0, r0, t0, h1, r1, t1, out_v, sem0, sem1):
        wid = lax.axis_index("s") * NUM_CORES + lax.axis_index("c")
        row0 = wid * n_chunks
        pltpu.sync_copy(hrow_hbm.at[pl.ds(row0, n_chunks)], hrow_v)
        pltpu.sync_copy(hcol_hbm.at[pl.ds(row0, n_chunks)], hcol_v)
        pltpu.sync_copy(rrow_hbm.at[pl.ds(row0, n_chunks)], rrow_v)
        pltpu.sync_copy(rcol_hbm.at[pl.ds(row0, n_chunks)], rcol_v)
        pltpu.sync_copy(trow_hbm.at[pl.ds(row0, n_chunks)], trow_v)
        pltpu.sync_copy(tcol_hbm.at[pl.ds(row0, n_chunks)], tcol_v)

        lane = jnp.arange(LANES, dtype=jnp.int32)
        bufs = ((h0, r0, t0, sem0), (h1, r1, t1, sem1))

        def issue(j, buf):
            h_b, r_b, t_b, sem = buf
            pltpu.async_copy(ent_hbm.at[hrow_v.at[j]], h_b, sem)
            pltpu.async_copy(rel_hbm.at[rrow_v.at[j]], r_b, sem)
            pltpu.async_copy(ent_hbm.at[trow_v.at[j]], t_b, sem)

        def drain_compute(j, buf):
            h_b, r_b, t_b, sem = buf
            pltpu.make_async_copy(ent_hbm.at[hrow_v.at[j]], h_b, sem).wait()
            pltpu.make_async_copy(rel_hbm.at[rrow_v.at[j]], r_b, sem).wait()
            pltpu.make_async_copy(ent_hbm.at[trow_v.at[j]], t_b, sem).wait()

            for g in range(groups):
                rows = g * LANES + lane
                hc = hcol_v[j, pl.ds(g * LANES, LANES)]
                rc = rcol_v[j, pl.ds(g * LANES, LANES)]
                tc = tcol_v[j, pl.ds(g * LANES, LANES)]

                def d_body(d, acc, rows=rows, hc=hc, rc=rc, tc=tc):
                    col = jnp.full((LANES,), d, dtype=jnp.int32)
                    hv = plsc.load_gather(h_b, [rows, hc + col])
                    rv = plsc.load_gather(r_b, [rows, rc + col])
                    tv = plsc.load_gather(t_b, [rows, tc + col])
                    return acc + jnp.abs(hv + rv - tv)

                acc = lax.fori_loop(
                    0, DIM, d_body, jnp.zeros((LANES,), jnp.float32))
                out_v[pl.ds(j * CHUNK + g * LANES, LANES)] = acc

        issue(0, bufs[0])
        for j in range(n_chunks):
            if j + 1 < n_chunks:
                issue(j + 1, bufs[(j + 1) % 2])
            drain_compute(j, bufs[j % 2])

        pltpu.sync_copy(out_v, out_hbm.at[pl.ds(wid * per_w, per_w)])

    return k


def kernel(positive_triplets, negative_triplets, entities_emb, relations_emb):
    batch = positive_triplets.shape[0]
    total = 2 * batch
    trip = jnp.concatenate(
        [positive_triplets, negative_triplets], axis=0).astype(jnp.int32)
    n_rows = total // CHUNK

    one = jnp.where(jnp.isfinite(entities_emb[0, 0]), jnp.float32(1.0),
                    jnp.float32(2.0))
    ent_lin = jnp.reshape(
        entities_emb, (entities_emb.shape[0] // 2, PAD)) * one
    rel_lin = jnp.reshape(
        relations_emb, (relations_emb.shape[0] // 2, PAD)) * one

    def split(col):
        return ((col // 2).reshape(n_rows, CHUNK),
                (DIM * (col % 2)).reshape(n_rows, CHUNK))

    hrow, hcol = split(trip[:, 0])
    rrow, rcol = split(trip[:, 1])
    trow, tcol = split(trip[:, 2])

    out = _sc_gather(total)(
        hrow, hcol, rrow, rcol, trow, tcol, ent_lin, rel_lin)
    return out[:batch], out[batch:]


# speedup vs baseline: 1.6161x; 1.6161x over previous
import functools

import jax
import jax.numpy as jnp
from jax import lax
from jax.experimental import pallas as pl
from jax.experimental.pallas import tpu as pltpu
from jax.experimental.pallas import tpu_sc as plsc

DIM = 64
LANES = 16
NUM_CORES = 2
NUM_SUBCORES = 16
NUM_WORKERS = NUM_CORES * NUM_SUBCORES
CHUNK = 128

_MESH = plsc.VectorSubcoreMesh(
    core_axis_name="c", subcore_axis_name="s",
    num_cores=NUM_CORES, num_subcores=NUM_SUBCORES)

_PARAMS = pltpu.CompilerParams(
    needs_layout_passes=False, use_tc_tiling_on_sc=False)


def _sc_diff_ht(total):
    per_w = total // NUM_WORKERS
    n_chunks = per_w // CHUNK

    @functools.partial(
        pl.kernel,
        out_type=jax.ShapeDtypeStruct((total, DIM), jnp.float32),
        mesh=_MESH,
        compiler_params=_PARAMS,
        scratch_types=[
            pltpu.VMEM((n_chunks, CHUNK), jnp.int32),
            pltpu.VMEM((n_chunks, CHUNK), jnp.int32),
            pltpu.VMEM((CHUNK, DIM), jnp.float32),
            pltpu.VMEM((CHUNK, DIM), jnp.float32),
            pltpu.VMEM((CHUNK, DIM), jnp.float32),
            pltpu.VMEM((CHUNK, DIM), jnp.float32),
            pltpu.VMEM((CHUNK, DIM), jnp.float32),
            pltpu.SemaphoreType.DMA,
            pltpu.SemaphoreType.DMA,
        ],
    )
    def k(hidx_hbm, tidx_hbm, ent_hbm, out_hbm,
          hidx_v, tidx_v, h0, t0, h1, t1, d_v, sem0, sem1):
        wid = lax.axis_index("s") * NUM_CORES + lax.axis_index("c")
        row0 = wid * n_chunks
        pltpu.sync_copy(hidx_hbm.at[pl.ds(row0, n_chunks)], hidx_v)
        pltpu.sync_copy(tidx_hbm.at[pl.ds(row0, n_chunks)], tidx_v)

        bufs = ((h0, t0, sem0), (h1, t1, sem1))

        def issue(j, buf):
            h_b, t_b, sem = buf
            pltpu.async_copy(ent_hbm.at[hidx_v.at[j]], h_b, sem)
            pltpu.async_copy(ent_hbm.at[tidx_v.at[j]], t_b, sem)

        def drain_compute(j, buf):
            h_b, t_b, sem = buf
            pltpu.make_async_copy(ent_hbm.at[hidx_v.at[j]], h_b, sem).wait()
            pltpu.make_async_copy(ent_hbm.at[tidx_v.at[j]], t_b, sem).wait()

            def row_body(i, _):
                for c in range(DIM // LANES):
                    sl = pl.ds(c * LANES, LANES)
                    d_v[i, sl] = h_b[i, sl] - t_b[i, sl]
                return 0

            lax.fori_loop(0, CHUNK, row_body, 0)
            pltpu.sync_copy(
                d_v, out_hbm.at[pl.ds((row0 + j) * CHUNK, CHUNK)])

        issue(0, bufs[0])
        for j in range(n_chunks):
            if j + 1 < n_chunks:
                issue(j + 1, bufs[(j + 1) % 2])
            drain_compute(j, bufs[j % 2])

    return k


def _sc_score(total):
    per_w = total // NUM_WORKERS
    n_chunks = per_w // CHUNK
    groups = CHUNK // LANES

    @functools.partial(
        pl.kernel,
        out_type=jax.ShapeDtypeStruct((total,), jnp.float32),
        mesh=_MESH,
        compiler_params=_PARAMS,
        scratch_types=[
            pltpu.VMEM((n_chunks, CHUNK), jnp.int32),
            pltpu.VMEM((CHUNK, DIM), jnp.float32),
            pltpu.VMEM((CHUNK, DIM), jnp.float32),
            pltpu.VMEM((CHUNK, DIM), jnp.float32),
            pltpu.VMEM((CHUNK, DIM), jnp.float32),
            pltpu.VMEM((per_w,), jnp.float32),
            pltpu.SemaphoreType.DMA,
            pltpu.SemaphoreType.DMA,
        ],
    )
    def k(ridx_hbm, rel_hbm, d_hbm, out_hbm,
          ridx_v, r0, d0, r1, d1, out_v, sem0, sem1):
        wid = lax.axis_index("s") * NUM_CORES + lax.axis_index("c")
        row0 = wid * n_chunks
        pltpu.sync_copy(ridx_hbm.at[pl.ds(row0, n_chunks)], ridx_v)

        lane = jnp.arange(LANES, dtype=jnp.int32)
        bufs = ((r0, d0, sem0), (r1, d1, sem1))

        def issue(j, buf):
            r_b, d_b, sem = buf
            pltpu.async_copy(rel_hbm.at[ridx_v.at[j]], r_b, sem)
            pltpu.async_copy(
                d_hbm.at[pl.ds((row0 + j) * CHUNK, CHUNK)], d_b, sem)

        def drain_compute(j, buf):
            r_b, d_b, sem = buf
            pltpu.make_async_copy(rel_hbm.at[ridx_v.at[j]], r_b, sem).wait()
            pltpu.make_async_copy(
                d_hbm.at[pl.ds((row0 + j) * CHUNK, CHUNK)], d_b, sem).wait()

            for g in range(groups):
                rows = g * LANES + lane

                def d_body(d, acc, rows=rows):
                    col = jnp.full((LANES,), d, dtype=jnp.int32)
                    rv = plsc.load_gather(r_b, [rows, col])
                    dv = plsc.load_gather(d_b, [rows, col])
                    return acc + jnp.abs(dv + rv)

                acc = lax.fori_loop(
                    0, DIM, d_body, jnp.zeros((LANES,), jnp.float32))
                out_v[pl.ds(j * CHUNK + g * LANES, LANES)] = acc

        issue(0, bufs[0])
        for j in range(n_chunks):
            if j + 1 < n_chunks:
                issue(j + 1, bufs[(j + 1) % 2])
            drain_compute(j, bufs[j % 2])

        pltpu.sync_copy(out_v, out_hbm.at[pl.ds(wid * per_w, per_w)])

    return k


def kernel(positive_triplets, negative_triplets, entities_emb, relations_emb):
    batch = positive_triplets.shape[0]
    total = 2 * batch
    trip = jnp.concatenate(
        [positive_triplets, negative_triplets], axis=0).astype(jnp.int32)
    n_rows = total // CHUNK
    hidx = trip[:, 0].reshape(n_rows, CHUNK)
    ridx = trip[:, 1].reshape(n_rows, CHUNK)
    tidx = trip[:, 2].reshape(n_rows, CHUNK)

    d_ht = _sc_diff_ht(total)(hidx, tidx, entities_emb)
    out = _sc_score(total)(ridx, relations_emb, d_ht)
    return out[:batch], out[batch:]


# speedup vs baseline: 2.3461x; 1.4517x over previous
import functools

import jax
import jax.numpy as jnp
from jax import lax
from jax.experimental import pallas as pl
from jax.experimental.pallas import tpu as pltpu
from jax.experimental.pallas import tpu_sc as plsc

DIM = 64
LANES = 16
NUM_CORES = 2
NUM_SUBCORES = 16
NUM_WORKERS = NUM_CORES * NUM_SUBCORES
K = 32
NSEM = 6


def _sc_transe(total):
    per_w = total // NUM_WORKERS
    n_batches = per_w // K
    assert n_batches % 2 == 0
    assert (3 * K) % NSEM == 0

    mesh = plsc.VectorSubcoreMesh(
        core_axis_name="c", subcore_axis_name="s",
        num_cores=NUM_CORES, num_subcores=NUM_SUBCORES)

    @functools.partial(
        pl.kernel,
        out_type=jax.ShapeDtypeStruct((total,), jnp.float32),
        mesh=mesh,
        compiler_params=pltpu.CompilerParams(needs_layout_passes=False),
        scratch_types=[
            pltpu.VMEM((per_w,), jnp.int32),
            pltpu.VMEM((per_w,), jnp.int32),
            pltpu.VMEM((per_w,), jnp.int32),
            pltpu.VMEM((K, DIM), jnp.float32),
            pltpu.VMEM((K, DIM), jnp.float32),
            pltpu.VMEM((K, DIM), jnp.float32),
            pltpu.VMEM((K, DIM), jnp.float32),
            pltpu.VMEM((K, DIM), jnp.float32),
            pltpu.VMEM((K, DIM), jnp.float32),
            pltpu.VMEM((per_w,), jnp.float32),
            [pltpu.SemaphoreType.DMA] * NSEM,
            [pltpu.SemaphoreType.DMA] * NSEM,
        ],
    )
    def k(hidx_hbm, ridx_hbm, tidx_hbm, ent_hbm, rel_hbm, out_hbm,
          hidx_v, ridx_v, tidx_v, h0, r0, t0, h1, r1, t1, out_v,
          sems0, sems1):
        wid = lax.axis_index("s") * NUM_CORES + lax.axis_index("c")
        base = wid * per_w
        pltpu.sync_copy(hidx_hbm.at[pl.ds(base, per_w)], hidx_v)
        pltpu.sync_copy(ridx_hbm.at[pl.ds(base, per_w)], ridx_v)
        pltpu.sync_copy(tidx_hbm.at[pl.ds(base, per_w)], tidx_v)

        lane = jnp.arange(LANES, dtype=jnp.int32)
        bufs = ((h0, r0, t0, sems0), (h1, r1, t1, sems1))

        def issue(b, buf):
            h_b, r_b, t_b, sems = buf
            b0 = b * K
            n = 0
            for g in range(K // LANES):
                hvec = hidx_v[pl.ds(b0 + g * LANES, LANES)]
                rvec = ridx_v[pl.ds(b0 + g * LANES, LANES)]
                tvec = tidx_v[pl.ds(b0 + g * LANES, LANES)]
                for i in range(LANES):
                    slot = g * LANES + i
                    pltpu.async_copy(ent_hbm.at[pl.ds(hvec[i], 1)],
                                     h_b.at[pl.ds(slot, 1)], sems[n % NSEM])
                    n += 1
                    pltpu.async_copy(rel_hbm.at[pl.ds(rvec[i], 1)],
                                     r_b.at[pl.ds(slot, 1)], sems[n % NSEM])
                    n += 1
                    pltpu.async_copy(ent_hbm.at[pl.ds(tvec[i], 1)],
                                     t_b.at[pl.ds(slot, 1)], sems[n % NSEM])
                    n += 1

        def drain_compute(b, buf):
            h_b, r_b, t_b, sems = buf
            rows_per_sem = (3 * K) // NSEM
            for s in range(NSEM):
                pltpu.make_async_copy(
                    ent_hbm.at[pl.ds(0, rows_per_sem)],
                    h_b.at[pl.ds(0, rows_per_sem)], sems[s]).wait()
            for g in range(K // LANES):
                rows = g * LANES + lane

                def d_body(d, acc, rows=rows):
                    col = jnp.full((LANES,), d, dtype=jnp.int32)
                    hv = plsc.load_gather(h_b, [rows, col])
                    rv = plsc.load_gather(r_b, [rows, col])
                    tv = plsc.load_gather(t_b, [rows, col])
                    return acc + jnp.abs(hv + rv - tv)

                acc = lax.fori_loop(
                    0, DIM, d_body, jnp.zeros((LANES,), jnp.float32))
                out_v[pl.ds(b * K + g * LANES, LANES)] = acc

        issue(0, bufs[0])

        def pair_body(p, _):
            b = p * 2
            issue(b + 1, bufs[1])
            drain_compute(b, bufs[0])

            @pl.when(b + 2 < n_batches)
            def _():
                issue(b + 2, bufs[0])

            drain_compute(b + 1, bufs[1])
            return 0

        lax.fori_loop(0, n_batches // 2, pair_body, 0)

        pltpu.sync_copy(out_v, out_hbm.at[pl.ds(base, per_w)])

    return k


def kernel(positive_triplets, negative_triplets, entities_emb, relations_emb):
    batch = positive_triplets.shape[0]
    total = 2 * batch
    trip = jnp.concatenate(
        [positive_triplets, negative_triplets], axis=0).astype(jnp.int32)

    out = _sc_transe(total)(
        trip[:, 0], trip[:, 1], trip[:, 2], entities_emb, relations_emb)
    return out[:batch], out[batch:]
